# baseline (speedup 1.0000x reference)
import jax
import jax.numpy as jnp
from jax import lax
from jax.experimental import pallas as pl
from jax.experimental.pallas import tpu as pltpu
from jax.experimental.pallas import tpu_sc as plsc

B, N, H, K = 4, 512, 32, 22
NC, NS, L = 2, 16, 16
NW = NC * NS
PAIRS = B * N
PPW = PAIRS // NW
C = 2
CN = C * N
STEPS = PPW // C
SLOTS = 2


def _sc_body(sp_hbm, tbl_hbm, out_hbm, tbl_v, dbuf, obuf, isems, osems):
    wid = lax.axis_index("s") * NC + lax.axis_index("c")
    pltpu.sync_copy(tbl_hbm, tbl_v)
    pair0 = wid * PPW
    b = pair0 // N
    i_base = pair0 - b * N

    def start_in(step, sl):
        pltpu.async_copy(
            sp_hbm.at[b, pl.ds(i_base + step * C, C), :],
            dbuf.at[sl], isems.at[sl])

    def start_out(step, sl):
        pltpu.async_copy(
            obuf.at[sl],
            out_hbm.at[b, :, pl.ds(i_base + step * C, C), :],
            osems.at[sl])

    def wait_in(sl):
        pltpu.make_async_copy(sp_hbm.at[0, pl.ds(0, C), :], dbuf.at[sl],
                              isems.at[sl]).wait()

    def wait_out(sl):
        pltpu.make_async_copy(obuf.at[sl],
                              out_hbm.at[0, :, pl.ds(0, C), :],
                              osems.at[sl]).wait()

    lane = jnp.arange(L, dtype=jnp.int32)

    def compute(sl):
        @plsc.parallel_loop(0, CN // L, unroll=8)
        def _(t):
            c, j = divmod(t * L, N)
            d16 = dbuf[sl, c, pl.ds(j, L)]
            d16 = jnp.minimum(jnp.maximum(d16, 0), K - 1)
            d16s = d16 * L + lane
            for h in range(H):
                obuf[sl, h, c, pl.ds(j, L)] = plsc.load_gather(
                    tbl_v.at[pl.ds(h * (K * L), K * L)], [d16s])

    for sl in range(SLOTS):
        start_in(sl, sl)

    def step_group(g, carry):
        for sl in range(SLOTS):
            step = g * SLOTS + sl
            wait_in(sl)

            @pl.when(g > 0)
            def _():
                wait_out(sl)

            compute(sl)
            start_out(step, sl)

            @pl.when(step + SLOTS < STEPS)
            def _():
                start_in(step + SLOTS, sl)
        return carry

    lax.fori_loop(0, STEPS // SLOTS, step_group, None)
    for sl in range(SLOTS):
        wait_out(sl)


@jax.jit
def kernel(sp_dist, table):
    tflat = jnp.transpose(table).reshape(-1)
    trep = jnp.repeat(tflat, L)
    mesh = plsc.VectorSubcoreMesh(core_axis_name="c", subcore_axis_name="s")
    run = pl.kernel(
        _sc_body,
        out_type=jax.ShapeDtypeStruct((B, H, N, N), jnp.float32),
        mesh=mesh,
        scratch_types=[
            pltpu.VMEM((H * K * L,), jnp.float32),
            pltpu.VMEM((SLOTS, C, N), jnp.int32),
            pltpu.VMEM((SLOTS, H, C, N), jnp.float32),
            pltpu.SemaphoreType.DMA((SLOTS,)),
            pltpu.SemaphoreType.DMA((SLOTS,)),
        ],
        compiler_params=pltpu.CompilerParams(needs_layout_passes=False),
    )
    return run(sp_dist, trep)


# speedup vs baseline: 1.0146x; 1.0146x over previous
import jax
import jax.numpy as jnp
from jax import lax
from jax.experimental import pallas as pl
from jax.experimental.pallas import tpu as pltpu
from jax.experimental.pallas import tpu_sc as plsc

B, N, H, K = 4, 512, 32, 22
NC, NS, L = 2, 16, 16
NW = NC * NS
PAIRS = B * N
PPW = PAIRS // NW
C = 2
CN = C * N
STEPS = PPW // C
SLOTS = 2


def _sc_body(sp_hbm, tbl_hbm, out_hbm, tbl_raw, tbl_v, dbuf, obuf, isems,
             osems):
    wid = lax.axis_index("s") * NC + lax.axis_index("c")
    pltpu.sync_copy(tbl_hbm, tbl_raw)
    lane0 = jnp.zeros((L,), dtype=jnp.int32)

    @plsc.parallel_loop(0, K * H, unroll=4)
    def _(v):
        src = (v % K) * H + v // K
        tbl_v[pl.ds(v * L, L)] = plsc.load_gather(tbl_raw, [lane0 + src])
    pair0 = wid * PPW
    b = pair0 // N
    i_base = pair0 - b * N

    def start_in(step, sl):
        pltpu.async_copy(
            sp_hbm.at[b, pl.ds(i_base + step * C, C), :],
            dbuf.at[sl], isems.at[sl])

    def start_out(step, sl):
        pltpu.async_copy(
            obuf.at[sl],
            out_hbm.at[b, :, pl.ds(i_base + step * C, C), :],
            osems.at[sl])

    def wait_in(sl):
        pltpu.make_async_copy(sp_hbm.at[0, pl.ds(0, C), :], dbuf.at[sl],
                              isems.at[sl]).wait()

    def wait_out(sl):
        pltpu.make_async_copy(obuf.at[sl],
                              out_hbm.at[0, :, pl.ds(0, C), :],
                              osems.at[sl]).wait()

    lane = jnp.arange(L, dtype=jnp.int32)

    def compute(sl):
        @plsc.parallel_loop(0, CN // L, unroll=4)
        def _(t):
            c, j = divmod(t * L, N)
            d16 = dbuf[sl, c, pl.ds(j, L)]
            d16 = jnp.minimum(jnp.maximum(d16, 0), K - 1)
            d16s = d16 * L + lane
            for h in range(H):
                obuf[sl, h, c, pl.ds(j, L)] = plsc.load_gather(
                    tbl_v.at[pl.ds(h * (K * L), K * L)], [d16s])

    for sl in range(SLOTS):
        start_in(sl, sl)

    def step_group(g, carry):
        for sl in range(SLOTS):
            step = g * SLOTS + sl
            wait_in(sl)

            @pl.when(g > 0)
            def _():
                wait_out(sl)

            compute(sl)
            start_out(step, sl)

            @pl.when(step + SLOTS < STEPS)
            def _():
                start_in(step + SLOTS, sl)
        return carry

    lax.fori_loop(0, STEPS // SLOTS, step_group, None)
    for sl in range(SLOTS):
        wait_out(sl)


@jax.jit
def kernel(sp_dist, table):
    tflat = table.reshape(-1)
    mesh = plsc.VectorSubcoreMesh(core_axis_name="c", subcore_axis_name="s")
    run = pl.kernel(
        _sc_body,
        out_type=jax.ShapeDtypeStruct((B, H, N, N), jnp.float32),
        mesh=mesh,
        scratch_types=[
            pltpu.VMEM((K * H,), jnp.float32),
            pltpu.VMEM((H * K * L,), jnp.float32),
            pltpu.VMEM((SLOTS, C, N), jnp.int32),
            pltpu.VMEM((SLOTS, H, C, N), jnp.float32),
            pltpu.SemaphoreType.DMA((SLOTS,)),
            pltpu.SemaphoreType.DMA((SLOTS,)),
        ],
        compiler_params=pltpu.CompilerParams(needs_layout_passes=False),
    )
    return run(sp_dist, tflat)
